# baseline (speedup 1.0000x reference)
import functools

import jax
import jax.numpy as jnp
from jax import lax
from jax.experimental import pallas as pl
from jax.experimental.pallas import tpu as pltpu
from jax.experimental.pallas import tpu_sc as plsc

_N_VOCAB = 100000
_D = 128
_SCALE = float(_D) ** 0.5

_NC = 2
_NS = 16
_NW = _NC * _NS
_B = 4096 * 200
_BPW = _B // _NW
_CHUNK = 128
_NCHUNK = _BPW // _CHUNK


def _scale_body(t_ref, o_ref):
    o_ref[...] = t_ref[...] * _SCALE


def _scale_table(table):
    grid = 25
    blk = _N_VOCAB // grid
    return pl.pallas_call(
        _scale_body,
        out_shape=jax.ShapeDtypeStruct((_N_VOCAB, _D), jnp.float32),
        grid=(grid,),
        in_specs=[pl.BlockSpec((blk, _D), lambda i: (i, 0))],
        out_specs=pl.BlockSpec((blk, _D), lambda i: (i, 0)),
    )(table)


_mesh = plsc.VectorSubcoreMesh(core_axis_name="c", subcore_axis_name="s")

_NBUF = 5


@functools.partial(
    pl.kernel,
    mesh=_mesh,
    out_type=jax.ShapeDtypeStruct((_NW, _NCHUNK, _CHUNK, _D), jnp.float32),
    scratch_types=[
        pltpu.VMEM((_NCHUNK, _CHUNK), jnp.int32),
        pltpu.VMEM((_NBUF, _CHUNK, _D), jnp.float32),
    ]
    + [pltpu.SemaphoreType.DMA] * (2 * _NBUF),
)
def _sc_gather(table_hbm, idx_hbm, out_hbm, idx_v, bufs, *sems):
    gsem = sems[:_NBUF]
    ssem = sems[_NBUF:]
    wid = lax.axis_index("s") * _NC + lax.axis_index("c")
    pltpu.sync_copy(idx_hbm.at[wid], idx_v)

    def fire_g(m, b):
        pltpu.async_copy(table_hbm.at[idx_v.at[m]], bufs.at[b], gsem[b])

    def wait_g(m, b):
        pltpu.make_async_copy(table_hbm.at[idx_v.at[m]], bufs.at[b], gsem[b]).wait()

    def scale_buf(b):
        def sbody(i, carry):
            for r in range(8):
                for j in range(_D // 16):
                    sl = (b, i * 8 + r, pl.ds(j * 16, 16))
                    bufs[sl] = bufs[sl] * _SCALE
            return carry

        lax.fori_loop(0, _CHUNK // 8, sbody, 0)

    def fire_s(m, b):
        scale_buf(b)
        pltpu.async_copy(bufs.at[b], out_hbm.at[wid, m], ssem[b])

    def wait_s(m, b):
        pltpu.make_async_copy(bufs.at[b], out_hbm.at[wid, m], ssem[b]).wait()

    for b in range(_NBUF - 1):
        fire_g(b, b)
    wait_g(0, 0)
    fire_s(0, 0)
    fire_g(_NBUF - 1, _NBUF - 1)

    def body(i, carry):
        for j in range(_NBUF):
            m = 1 + i * _NBUF + j
            b = (1 + j) % _NBUF
            bp = (b - 1) % _NBUF
            wait_g(m, b)
            fire_s(m, b)
            wait_s(m - 1, bp)
            fire_g(m + _NBUF - 1, bp)
        return carry

    n_steady = _NCHUNK - _NBUF + 1 - 1
    lax.fori_loop(0, n_steady // _NBUF, body, 0)

    for m in range(_NCHUNK - _NBUF + 1, _NCHUNK):
        b = m % _NBUF
        wait_g(m, b)
        fire_s(m, b)
        wait_s(m - 1, (b - 1) % _NBUF)
    wait_s(_NCHUNK - 1, (_NCHUNK - 1) % _NBUF)


def kernel(raw_src_seq, src_word_emb_weight):
    idx = raw_src_seq.astype(jnp.int32).reshape(_NW, _NCHUNK, _CHUNK)
    out = _sc_gather(src_word_emb_weight, idx)
    return out.reshape(4096, 200, _D)
